# Initial kernel scaffold; baseline (speedup 1.0000x reference)
import jax
import jax.numpy as jnp
from jax.experimental import pallas as pl


def kernel(x, edge_index, batch, W1, b1, W2, b2, W3, b3, Wm1, bm1, Wm2, bm2):
    raise NotImplementedError("write your pallas kernel here")


# speedup vs baseline: 9.6645x; 9.6645x over previous
import functools

import jax
import jax.numpy as jnp
from jax import lax
from jax.experimental import pallas as pl
from jax.experimental.pallas import tpu as pltpu
from jax.experimental.pallas import tpu_sc as plsc

_N = 10000
_E = 320000
_G = 16
_D = 128
_NC = 2
_NS = 16
_NW = _NC * _NS
_CH = 128
_K = 79
_EP = _NW * _K * _CH
_NP = 10112
_RT = _NP // _NS

_BR = 2528


@functools.lru_cache(maxsize=None)
def _build_sc_degree():
    mesh = plsc.VectorSubcoreMesh(
        core_axis_name="c", subcore_axis_name="s",
        num_cores=_NC, num_subcores=_NS)

    @functools.partial(
        pl.kernel,
        out_type=jax.ShapeDtypeStruct((_NC, _NP, _D), jnp.float32),
        mesh=mesh,
        scratch_types=[
            pltpu.VMEM_SHARED((_NP, _D), jnp.float32),
            pltpu.VMEM((_K, _CH), jnp.int32),
            pltpu.VMEM((_CH, _D), jnp.float32),
        ],
    )
    def deg_kernel(dst_hbm, ones_hbm, zeros_hbm, out_hbm, acc_sh, idx_v, ones_v):
        c = lax.axis_index("c")
        s = lax.axis_index("s")
        w = c * _NS + s
        r0 = s * _RT
        pltpu.sync_copy(zeros_hbm.at[pl.ds(r0, _RT)], acc_sh.at[pl.ds(r0, _RT)])
        pltpu.sync_copy(ones_hbm, ones_v)
        pltpu.sync_copy(dst_hbm.at[w], idx_v)
        plsc.subcore_barrier()

        def body(j, carry):
            pltpu.sync_copy(ones_v, acc_sh.at[idx_v.at[j]], add=True)
            return carry

        lax.fori_loop(0, _K, body, 0)
        plsc.subcore_barrier()
        pltpu.sync_copy(acc_sh.at[pl.ds(r0, _RT)], out_hbm.at[c, pl.ds(r0, _RT)])

    return deg_kernel


@functools.lru_cache(maxsize=None)
def _build_sc_scatter():
    mesh = plsc.VectorSubcoreMesh(
        core_axis_name="c", subcore_axis_name="s",
        num_cores=_NC, num_subcores=_NS)

    @functools.partial(
        pl.kernel,
        out_type=jax.ShapeDtypeStruct((_NC, _NP, _D), jnp.float32),
        mesh=mesh,
        scratch_types=[
            pltpu.VMEM_SHARED((_NP, _D), jnp.float32),
            pltpu.VMEM((_K, _CH), jnp.int32),
            pltpu.VMEM((_K, _CH), jnp.int32),
            pltpu.VMEM((_CH, _D), jnp.float32),
            pltpu.SemaphoreType.DMA,
        ],
    )
    def mp_kernel(g_hbm, src_hbm, dst_hbm, zeros_hbm, out_hbm,
                  acc_sh, src_v, dst_v, rows_v, sem):
        c = lax.axis_index("c")
        s = lax.axis_index("s")
        w = c * _NS + s
        r0 = s * _RT
        pltpu.sync_copy(zeros_hbm.at[pl.ds(r0, _RT)], acc_sh.at[pl.ds(r0, _RT)])
        pltpu.sync_copy(src_hbm.at[w], src_v)
        pltpu.sync_copy(dst_hbm.at[w], dst_v)
        plsc.subcore_barrier()

        def body(j, carry):
            pltpu.async_copy(g_hbm.at[src_v.at[j]], rows_v, sem).wait()
            pltpu.sync_copy(rows_v, acc_sh.at[dst_v.at[j]], add=True)
            return carry

        lax.fori_loop(0, _K, body, 0)
        plsc.subcore_barrier()
        pltpu.sync_copy(acc_sh.at[pl.ds(r0, _RT)], out_hbm.at[c, pl.ds(r0, _RT)])

    return mp_kernel


def _sc_degree(dst, ones, zeros):
    return _build_sc_degree()(dst, ones, zeros)


def _sc_scatter(g, src, dst, zeros128):
    return _build_sc_scatter()(g, src, dst, zeros128)


def _tc_prep_body(x_ref, degp_ref, w_ref, g_ref, dinv_ref):
    deg = degp_ref[0, :, 0:1] + degp_ref[1, :, 0:1] + 1.0
    dinv = lax.rsqrt(deg)
    h = jnp.dot(x_ref[...], w_ref[...], preferred_element_type=jnp.float32)
    g_ref[...] = h * dinv
    dinv_ref[...] = dinv


def _tc_prep(xp, degp, W1):
    grid = _NP // _BR
    return pl.pallas_call(
        _tc_prep_body,
        grid=(grid,),
        in_specs=[
            pl.BlockSpec((_BR, _D), lambda i: (i, 0)),
            pl.BlockSpec((_NC, _BR, _D), lambda i: (0, i, 0)),
            pl.BlockSpec((_D, _D), lambda i: (0, 0)),
        ],
        out_specs=[
            pl.BlockSpec((_BR, _D), lambda i: (i, 0)),
            pl.BlockSpec((_BR, 1), lambda i: (i, 0)),
        ],
        out_shape=[
            jax.ShapeDtypeStruct((_NP, _D), jnp.float32),
            jax.ShapeDtypeStruct((_NP, 1), jnp.float32),
        ],
    )(xp, degp, W1)


def _tc_layer_body(p_ref, g_ref, dinv_ref, b_ref, w_ref, gn_ref):
    dinv = dinv_ref[...]
    y = jnp.maximum(dinv * (p_ref[0] + p_ref[1] + g_ref[...]) + b_ref[...], 0.0)
    gn_ref[...] = jnp.dot(y, w_ref[...], preferred_element_type=jnp.float32) * dinv


def _tc_layer(p, g, dinv, b, Wn):
    grid = _NP // _BR
    return pl.pallas_call(
        _tc_layer_body,
        grid=(grid,),
        in_specs=[
            pl.BlockSpec((_NC, _BR, _D), lambda i: (0, i, 0)),
            pl.BlockSpec((_BR, _D), lambda i: (i, 0)),
            pl.BlockSpec((_BR, 1), lambda i: (i, 0)),
            pl.BlockSpec((1, _D), lambda i: (0, 0)),
            pl.BlockSpec((_D, _D), lambda i: (0, 0)),
        ],
        out_specs=pl.BlockSpec((_BR, _D), lambda i: (i, 0)),
        out_shape=jax.ShapeDtypeStruct((_NP, _D), jnp.float32),
    )(p, g, dinv, b, Wn)


def _tc_head_body(p_ref, g_ref, dinv_ref, b_ref, batch_ref, wm1_ref, bm1_ref,
                  wm2_ref, bm2_ref, out_ref, acc_ref, cnt_ref):
    i = pl.program_id(0)

    @pl.when(i == 0)
    def _():
        acc_ref[...] = jnp.zeros_like(acc_ref)
        cnt_ref[...] = jnp.zeros_like(cnt_ref)

    dinv = dinv_ref[...]
    y = jnp.maximum(dinv * (p_ref[0] + p_ref[1] + g_ref[...]) + b_ref[...], 0.0)
    m = (batch_ref[...] == lax.broadcasted_iota(jnp.int32, (_BR, _G), 1)
         ).astype(jnp.float32)
    dn = (((0,), (0,)), ((), ()))
    acc_ref[...] += lax.dot_general(m, y, dn, preferred_element_type=jnp.float32)
    cnt_ref[...] += lax.dot_general(m, jnp.ones((_BR, 1), jnp.float32), dn,
                                    preferred_element_type=jnp.float32)

    @pl.when(i == pl.num_programs(0) - 1)
    def _():
        pooled = acc_ref[...] / jnp.maximum(cnt_ref[...], 1.0)
        z = jnp.maximum(
            jnp.dot(pooled, wm1_ref[...], preferred_element_type=jnp.float32)
            + bm1_ref[...], 0.0)
        out_ref[...] = (
            jnp.dot(z, wm2_ref[...], preferred_element_type=jnp.float32)
            + bm2_ref[...])


def _tc_head(p, g, dinv, b, batch2d, Wm1, bm1, Wm2, bm2):
    grid = _NP // _BR
    return pl.pallas_call(
        _tc_head_body,
        grid=(grid,),
        in_specs=[
            pl.BlockSpec((_NC, _BR, _D), lambda i: (0, i, 0)),
            pl.BlockSpec((_BR, _D), lambda i: (i, 0)),
            pl.BlockSpec((_BR, 1), lambda i: (i, 0)),
            pl.BlockSpec((1, _D), lambda i: (0, 0)),
            pl.BlockSpec((_BR, 1), lambda i: (i, 0)),
            pl.BlockSpec((_D, _D // 2), lambda i: (0, 0)),
            pl.BlockSpec((1, _D // 2), lambda i: (0, 0)),
            pl.BlockSpec((_D // 2, 10), lambda i: (0, 0)),
            pl.BlockSpec((1, 10), lambda i: (0, 0)),
        ],
        out_specs=pl.BlockSpec((_G, 10), lambda i: (0, 0)),
        out_shape=jax.ShapeDtypeStruct((_G, 10), jnp.float32),
        scratch_shapes=[
            pltpu.VMEM((_G, _D), jnp.float32),
            pltpu.VMEM((_G, 1), jnp.float32),
        ],
    )(p, g, dinv, b, batch2d, Wm1, bm1, Wm2, bm2)


def kernel(x, edge_index, batch, W1, b1, W2, b2, W3, b3, Wm1, bm1, Wm2, bm2):
    pad = _EP - _E
    src = jnp.concatenate(
        [edge_index[0], jnp.zeros((pad,), jnp.int32)]).reshape(_NW, _K, _CH)
    dst = jnp.concatenate(
        [edge_index[1], jnp.full((pad,), _N, jnp.int32)]).reshape(_NW, _K, _CH)
    xp = jnp.pad(x, ((0, _NP - _N), (0, 0)))
    batch2d = jnp.concatenate(
        [batch, jnp.full((_NP - _N,), _G, jnp.int32)]).reshape(_NP, 1)
    zeros128 = jnp.zeros((_NP, _D), jnp.float32)
    ones128 = jnp.ones((_CH, _D), jnp.float32)

    degp = _sc_degree(dst, ones128, zeros128)
    g1, dinv = _tc_prep(xp, degp, W1)
    p1 = _sc_scatter(g1, src, dst, zeros128)
    g2 = _tc_layer(p1, g1, dinv, b1.reshape(1, _D), W2)
    p2 = _sc_scatter(g2, src, dst, zeros128)
    g3 = _tc_layer(p2, g2, dinv, b2.reshape(1, _D), W3)
    p3 = _sc_scatter(g3, src, dst, zeros128)
    return _tc_head(p3, g3, dinv, b3.reshape(1, _D), batch2d,
                    Wm1, bm1.reshape(1, _D // 2), Wm2, bm2.reshape(1, 10))
